# baseline (speedup 1.0000x reference)
import functools

import jax
import jax.numpy as jnp
from jax import lax
from jax.experimental import pallas as pl
from jax.experimental.pallas import tpu as pltpu
from jax.experimental.pallas import tpu_sc as plsc

BATCH = 16384
D = 64
NC = 2
NS = 16
NW = NC * NS
BPW = BATCH // NW
CHUNK = 128
NCH = BPW // CHUNK
GROUP = 16

_mesh = plsc.VectorSubcoreMesh(core_axis_name="c", subcore_axis_name="s")


@functools.partial(
    pl.kernel,
    mesh=_mesh,
    compiler_params=pltpu.CompilerParams(needs_layout_passes=False,
                                         use_tc_tiling_on_sc=False),
    out_type=jax.ShapeDtypeStruct((BATCH,), jnp.float32),
    scratch_types=[
        pltpu.VMEM((NCH, CHUNK), jnp.int32),
        pltpu.VMEM((NCH, CHUNK), jnp.int32),
        pltpu.VMEM((BPW, D), jnp.float32),
        pltpu.VMEM((BPW, D), jnp.float32),
        pltpu.VMEM((BPW,), jnp.float32),
        pltpu.SemaphoreType.DMA,
    ],
)
def _dot_kernel(uidx_hbm, iidx_hbm, utab_hbm, itab_hbm, out_hbm,
                uidx_v, iidx_v, urows_v, irows_v, out_v, sem):
    wid = lax.axis_index("s") * NC + lax.axis_index("c")
    base = wid * BPW

    for c in range(NCH):
        pltpu.sync_copy(uidx_hbm.at[pl.ds(base + c * CHUNK, CHUNK)],
                        uidx_v.at[c])
        pltpu.sync_copy(iidx_hbm.at[pl.ds(base + c * CHUNK, CHUNK)],
                        iidx_v.at[c])

    copies = []
    for c in range(NCH):
        copies.append(pltpu.async_copy(
            utab_hbm.at[uidx_v.at[c]],
            urows_v.at[pl.ds(c * CHUNK, CHUNK)], sem))
        copies.append(pltpu.async_copy(
            itab_hbm.at[iidx_v.at[c]],
            irows_v.at[pl.ds(c * CHUNK, CHUNK)], sem))
    for cp in copies:
        cp.wait()

    lanes = lax.iota(jnp.int32, 16)

    def group_body(g, _):
        row0 = g * GROUP
        total = jnp.zeros((16,), jnp.float32)
        for r in range(GROUP):
            row = row0 + r
            accs = []
            for c in range(D // 16):
                u = urows_v[row, pl.ds(c * 16, 16)]
                it = irows_v[row, pl.ds(c * 16, 16)]
                accs.append(u * it)
            acc = (accs[0] + accs[1]) + (accs[2] + accs[3])
            total = jnp.where(lanes == r, jnp.sum(acc), total)
        out_v[pl.ds(row0, GROUP)] = total
        return 0

    lax.fori_loop(0, BPW // GROUP, group_body, 0)

    pltpu.sync_copy(out_v, out_hbm.at[pl.ds(base, BPW)])


def kernel(user_indices, item_indices, user_table, item_table):
    out = _dot_kernel(user_indices.astype(jnp.int32),
                      item_indices.astype(jnp.int32),
                      user_table, item_table)
    return out.reshape(BATCH, 1)


# speedup vs baseline: 2.8457x; 2.8457x over previous
import functools

import jax
import jax.numpy as jnp
from jax import lax
from jax.experimental import pallas as pl
from jax.experimental.pallas import tpu as pltpu
from jax.experimental.pallas import tpu_sc as plsc

BATCH = 16384
D = 64
NROWS = 1000000
NC = 2
NS = 16
NW = NC * NS
BPW = BATCH // NW
CHW = 4
CW = CHW * 128
SB = 128
TAIL = (NROWS // 128) * 128 - CW + 128
TAILROW = (NROWS // 128) * 128

_mesh = plsc.VectorSubcoreMesh(core_axis_name="c", subcore_axis_name="s")


@functools.partial(
    pl.kernel,
    mesh=_mesh,
    compiler_params=pltpu.CompilerParams(needs_layout_passes=False),
    out_type=(jax.ShapeDtypeStruct((BATCH, 128), jnp.float32),
              jax.ShapeDtypeStruct((BATCH, 128), jnp.float32)),
    scratch_types=[
        pltpu.VMEM((BPW + 16,), jnp.int32),
        pltpu.VMEM((BPW // SB, SB), jnp.int32),
        pltpu.VMEM((D, CW), jnp.float32),
        pltpu.VMEM((D, 128), jnp.float32),
        pltpu.VMEM((SB, 128), jnp.float32),
        pltpu.SemaphoreType.DMA,
        pltpu.SemaphoreType.DMA,
    ],
)
def _gather_kernel(suidx_hbm, uperm_hbm, siidx_hbm, iperm_hbm,
                   utab_hbm, itab_hbm, utail_hbm, itail_hbm,
                   uscr_hbm, iscr_hbm,
                   seg_v, perm_v, chunk_v, tail_v, rows_v, sem1, sem2):
    wid = lax.axis_index("s") * NC + lax.axis_index("c")
    base = wid * BPW
    dvecs = [c * 16 + lax.iota(jnp.int32, 16) for c in range(D // 16)]

    def process_table(sidx_hbm, perm_hbm, tab3_hbm, tail_hbm, scr_hbm):
        pltpu.sync_copy(sidx_hbm.at[pl.ds(base, BPW)],
                        seg_v.at[pl.ds(0, BPW)])
        for j in range(BPW // SB):
            pltpu.sync_copy(perm_hbm.at[pl.ds(base + j * SB, SB)],
                            perm_v.at[j])
        pltpu.sync_copy(tail_hbm, tail_v)

        def extract(p, u, cstart):
            slot = p & (SB - 1)
            @pl.when(u < TAILROW)
            def _():
                col = jnp.full((16,), u - cstart, jnp.int32)
                for c in range(D // 16):
                    rows_v[slot, pl.ds(c * 16, 16)] = plsc.load_gather(
                        chunk_v, [dvecs[c], col])

            @pl.when(u >= TAILROW)
            def _():
                colt = jnp.full((16,), u - TAILROW, jnp.int32)
                for c in range(D // 16):
                    rows_v[slot, pl.ds(c * 16, 16)] = plsc.load_gather(
                        tail_v, [dvecs[c], colt])

            @pl.when(slot == SB - 1)
            def _():
                pltpu.async_copy(rows_v, scr_hbm.at[perm_v.at[p >> 7]],
                                 sem2).wait()

        def inner_body(carry):
            p, u, cstart = carry
            extract(p, u, cstart)
            nxt = seg_v[pl.ds(p + 1, 16)]
            return p + 1, nxt[0], cstart

        def inner_cond(carry):
            p, u, cstart = carry
            return (p < BPW) & ((u < cstart + CW) | (u >= TAILROW))

        def outer_body(carry):
            p, u = carry
            cstart = jnp.minimum(u >> 7, (TAIL >> 7)) * 128
            cps = [pltpu.async_copy(
                tab3_hbm.at[tr, :, pl.ds(cstart, CW)],
                chunk_v.at[pl.ds(tr * 8, 8), :], sem1) for tr in range(8)]
            for cp in cps:
                cp.wait()
            p, u, _ = lax.while_loop(inner_cond, inner_body,
                                     (p, u, cstart))
            return p, u

        def outer_cond(carry):
            p, _ = carry
            return p < BPW

        u0 = seg_v[pl.ds(0, 16)][0]
        lax.while_loop(outer_cond, outer_body, (jnp.int32(0), u0))

    process_table(suidx_hbm, uperm_hbm, utab_hbm, utail_hbm, uscr_hbm)
    process_table(siidx_hbm, iperm_hbm, itab_hbm, itail_hbm, iscr_hbm)


def _dot_body(u_ref, i_ref, o_ref):
    prod = u_ref[...] * i_ref[...]
    col = lax.broadcasted_iota(jnp.int32, prod.shape, 1)
    prod = jnp.where(col < D, prod, 0.0)
    o_ref[...] = jnp.sum(prod, axis=1)


_dot_tc = pl.pallas_call(
    _dot_body,
    grid=(8,),
    in_specs=[pl.BlockSpec((BATCH // 8, 128), lambda i: (i, 0)),
              pl.BlockSpec((BATCH // 8, 128), lambda i: (i, 0))],
    out_specs=pl.BlockSpec((BATCH // 8,), lambda i: (i,)),
    out_shape=jax.ShapeDtypeStruct((BATCH,), jnp.float32),
)


def kernel(user_indices, item_indices, user_table, item_table):
    ui = user_indices.astype(jnp.int32)
    ii = item_indices.astype(jnp.int32)
    pos = lax.iota(jnp.int32, BATCH)
    su, pu = lax.sort_key_val(ui, pos)
    si, pi = lax.sort_key_val(ii, pos)
    utab3 = user_table.T.reshape(8, 8, NROWS)
    itab3 = item_table.T.reshape(8, 8, NROWS)
    utail = jnp.pad(user_table[TAILROW:].T, ((0, 0), (0, 128 - (NROWS - TAILROW))))
    itail = jnp.pad(item_table[TAILROW:].T, ((0, 0), (0, 128 - (NROWS - TAILROW))))
    uscr, iscr = _gather_kernel(su, pu, si, pi, utab3, itab3, utail, itail)
    out = _dot_tc(uscr, iscr)
    return out.reshape(BATCH, 1)


# speedup vs baseline: 3.7876x; 1.3310x over previous
import functools

import jax
import jax.numpy as jnp
from jax import lax
from jax.experimental import pallas as pl
from jax.experimental.pallas import tpu as pltpu
from jax.experimental.pallas import tpu_sc as plsc

BATCH = 16384
D = 64
NROWS = 1000000
NC = 2
NS = 16
NW = NC * NS
BPW = BATCH // NW
CHW = 4
CW = CHW * 128
SB = 128
TAIL = (NROWS // 128) * 128 - CW + 128
TAILROW = (NROWS // 128) * 128

_mesh = plsc.VectorSubcoreMesh(core_axis_name="c", subcore_axis_name="s")


@functools.partial(
    pl.kernel,
    mesh=_mesh,
    compiler_params=pltpu.CompilerParams(needs_layout_passes=False),
    out_type=(jax.ShapeDtypeStruct((BATCH, 128), jnp.float32),
              jax.ShapeDtypeStruct((BATCH, 128), jnp.float32)),
    scratch_types=[
        pltpu.VMEM((BPW + 16,), jnp.int32),
        pltpu.VMEM((BPW // SB, SB), jnp.int32),
        pltpu.VMEM((D, 2 * CW), jnp.float32),
        pltpu.VMEM((D, 128), jnp.float32),
        pltpu.VMEM((SB, 128), jnp.float32),
        pltpu.SemaphoreType.DMA,
        pltpu.SemaphoreType.DMA,
        pltpu.SemaphoreType.DMA,
    ],
)
def _gather_kernel(suidx_hbm, uperm_hbm, siidx_hbm, iperm_hbm,
                   utab_hbm, itab_hbm, utail_hbm, itail_hbm,
                   uscr_hbm, iscr_hbm,
                   seg_v, perm_v, chunk_v, tail_v, rows_v, semA, semB, sem2):
    wid = lax.axis_index("s") * NC + lax.axis_index("c")
    base = wid * BPW
    dvecs = [c * 16 + lax.iota(jnp.int32, 16) for c in range(D // 16)]

    def process_table(sidx_hbm, perm_hbm, tab3_hbm, tail_hbm, scr_hbm):
        pltpu.sync_copy(sidx_hbm.at[pl.ds(base, BPW)],
                        seg_v.at[pl.ds(0, BPW)])
        for j in range(BPW // SB):
            pltpu.sync_copy(perm_hbm.at[pl.ds(base + j * SB, SB)],
                            perm_v.at[j])
        pltpu.sync_copy(tail_hbm, tail_v)

        def extract(p, u, cstart, half):
            slot = p & (SB - 1)
            @pl.when(u < TAILROW)
            def _():
                col = jnp.full((16,), half * CW + (u - cstart), jnp.int32)
                for c in range(D // 16):
                    rows_v[slot, pl.ds(c * 16, 16)] = plsc.load_gather(
                        chunk_v, [dvecs[c], col])

            @pl.when(u >= TAILROW)
            def _():
                colt = jnp.full((16,), u - TAILROW, jnp.int32)
                for c in range(D // 16):
                    rows_v[slot, pl.ds(c * 16, 16)] = plsc.load_gather(
                        tail_v, [dvecs[c], colt])

            @pl.when(slot == SB - 1)
            def _():
                pltpu.async_copy(rows_v, scr_hbm.at[perm_v.at[p >> 7]],
                                 sem2).wait()

        def fetch(win, half, sem):
            cstart = jnp.minimum(win, jnp.int32(TAIL >> 7)) * 128
            for tr in range(8):
                pltpu.async_copy(
                    tab3_hbm.at[tr, :, pl.ds(cstart, CW)],
                    chunk_v.at[pl.ds(tr * 8, 8), pl.ds(half * CW, CW)], sem)
            return cstart

        def drain(sem):
            for tr in range(8):
                pltpu.make_async_copy(
                    tab3_hbm.at[tr, :, pl.ds(0, CW)],
                    chunk_v.at[pl.ds(tr * 8, 8), pl.ds(0, CW)], sem).wait()

        def run_window(p, u, win, half):
            cstart = jnp.minimum(win, jnp.int32(TAIL >> 7)) * 128
            cend = cstart + CW

            def inner_body(carry):
                pp, uu = carry
                extract(pp, uu, cstart, half)
                nxt = seg_v[pl.ds(pp + 1, 16)]
                return pp + 1, nxt[0]

            def inner_cond(carry):
                pp, uu = carry
                return (pp < BPW) & ((uu < cend) | (uu >= TAILROW))

            return lax.while_loop(inner_cond, inner_body, (p, u))

        head = seg_v[pl.ds(0, 16)]
        u0 = head[0]
        u_last = seg_v[pl.ds(BPW - 16, 16)][15]
        tc_lo = jnp.minimum(u0 >> 7, jnp.int32(TAIL >> 7))
        span_hi = jnp.minimum(u_last >> 7, jnp.int32(TAILROW // 128 - 1))
        nchunks = jnp.maximum((span_hi - tc_lo + CHW) // CHW, 1)
        npairs = (nchunks + 1) // 2

        fetch(tc_lo, 0, semA)
        fetch(tc_lo + CHW, 1, semB)

        def pair_body(k2, carry):
            p, u = carry
            drain(semA)
            p, u = run_window(p, u, tc_lo + (2 * k2) * CHW, 0)
            fetch(tc_lo + (2 * k2 + 2) * CHW, 0, semA)
            drain(semB)
            p, u = run_window(p, u, tc_lo + (2 * k2 + 1) * CHW, 1)
            fetch(tc_lo + (2 * k2 + 3) * CHW, 1, semB)
            return p, u

        lax.fori_loop(0, npairs, pair_body, (jnp.int32(0), u0))
        drain(semA)
        drain(semB)

    process_table(suidx_hbm, uperm_hbm, utab_hbm, utail_hbm, uscr_hbm)
    process_table(siidx_hbm, iperm_hbm, itab_hbm, itail_hbm, iscr_hbm)


def _dot_body(u_ref, i_ref, o_ref):
    prod = u_ref[...] * i_ref[...]
    col = lax.broadcasted_iota(jnp.int32, prod.shape, 1)
    prod = jnp.where(col < D, prod, 0.0)
    o_ref[...] = jnp.sum(prod, axis=1)


_dot_tc = pl.pallas_call(
    _dot_body,
    grid=(8,),
    in_specs=[pl.BlockSpec((BATCH // 8, 128), lambda i: (i, 0)),
              pl.BlockSpec((BATCH // 8, 128), lambda i: (i, 0))],
    out_specs=pl.BlockSpec((BATCH // 8,), lambda i: (i,)),
    out_shape=jax.ShapeDtypeStruct((BATCH,), jnp.float32),
)


def kernel(user_indices, item_indices, user_table, item_table):
    ui = user_indices.astype(jnp.int32)
    ii = item_indices.astype(jnp.int32)
    pos = lax.iota(jnp.int32, BATCH)
    su, pu = lax.sort_key_val(ui, pos)
    si, pi = lax.sort_key_val(ii, pos)
    utab3 = user_table.T.reshape(8, 8, NROWS)
    itab3 = item_table.T.reshape(8, 8, NROWS)
    utail = jnp.pad(user_table[TAILROW:].T, ((0, 0), (0, 128 - (NROWS - TAILROW))))
    itail = jnp.pad(item_table[TAILROW:].T, ((0, 0), (0, 128 - (NROWS - TAILROW))))
    uscr, iscr = _gather_kernel(su, pu, si, pi, utab3, itab3, utail, itail)
    out = _dot_tc(uscr, iscr)
    return out.reshape(BATCH, 1)


# speedup vs baseline: 3.8605x; 1.0192x over previous
import functools

import jax
import jax.numpy as jnp
from jax import lax
from jax.experimental import pallas as pl
from jax.experimental.pallas import tpu as pltpu
from jax.experimental.pallas import tpu_sc as plsc

BATCH = 16384
D = 64
NROWS = 1000000
NC = 2
NS = 16
NW = NC * NS
BPW = BATCH // NW
CHW = 6
CW = CHW * 128
SB = 128
NTC = NROWS // 128
TAILC = NTC - CHW
TAILROW = NTC * 128

_mesh = plsc.VectorSubcoreMesh(core_axis_name="c", subcore_axis_name="s")


@functools.partial(
    pl.kernel,
    mesh=_mesh,
    compiler_params=pltpu.CompilerParams(needs_layout_passes=False),
    out_type=(jax.ShapeDtypeStruct((BATCH, 128), jnp.float32),
              jax.ShapeDtypeStruct((BATCH, 128), jnp.float32)),
    scratch_types=[
        pltpu.VMEM((BPW + 16,), jnp.int32),
        pltpu.VMEM((BPW // SB, SB), jnp.int32),
        pltpu.VMEM((8, 8, 2 * CW), jnp.float32),
        pltpu.VMEM((D, 128), jnp.float32),
        pltpu.VMEM((SB, 128), jnp.float32),
        pltpu.SemaphoreType.DMA,
        pltpu.SemaphoreType.DMA,
        pltpu.SemaphoreType.DMA,
    ],
)
def _gather_kernel(suidx_hbm, uperm_hbm, siidx_hbm, iperm_hbm,
                   utab_hbm, itab_hbm, utail_hbm, itail_hbm,
                   uscr_hbm, iscr_hbm,
                   seg_v, perm_v, chunk_v, tail_v, rows_v, semA, semB, sem2):
    wid = lax.axis_index("s") * NC + lax.axis_index("c")
    base = wid * BPW
    dvecs = [c * 16 + lax.iota(jnp.int32, 16) for c in range(D // 16)]
    bvecs = [(c * 16 + lax.iota(jnp.int32, 16)) >> 3 for c in range(D // 16)]
    svecs = [(c * 16 + lax.iota(jnp.int32, 16)) & 7 for c in range(D // 16)]

    def process_table(sidx_hbm, perm_hbm, tab3_hbm, tail_hbm, scr_hbm):
        pltpu.sync_copy(sidx_hbm.at[pl.ds(base, BPW)],
                        seg_v.at[pl.ds(0, BPW)])
        for j in range(BPW // SB):
            pltpu.sync_copy(perm_hbm.at[pl.ds(base + j * SB, SB)],
                            perm_v.at[j])
        pltpu.sync_copy(tail_hbm, tail_v)

        def extract(p, u, cstart, half):
            slot = p & (SB - 1)
            @pl.when(u < TAILROW)
            def _():
                col = jnp.full((16,), half * CW + (u - cstart), jnp.int32)
                for c in range(D // 16):
                    rows_v[slot, pl.ds(c * 16, 16)] = plsc.load_gather(
                        chunk_v, [bvecs[c], svecs[c], col])

            @pl.when(u >= TAILROW)
            def _():
                colt = jnp.full((16,), u - TAILROW, jnp.int32)
                for c in range(D // 16):
                    rows_v[slot, pl.ds(c * 16, 16)] = plsc.load_gather(
                        tail_v, [dvecs[c], colt])

            @pl.when(slot == SB - 1)
            def _():
                pltpu.async_copy(rows_v, scr_hbm.at[perm_v.at[p >> 7]],
                                 sem2).wait()

        def fetch(win, half, sem):
            cstart = jnp.minimum(win, jnp.int32(TAILC)) * 128
            pltpu.async_copy(
                tab3_hbm.at[:, :, pl.ds(cstart, CW)],
                chunk_v.at[:, :, pl.ds(half * CW, CW)], sem)

        def drain(sem):
            pltpu.make_async_copy(
                tab3_hbm.at[:, :, pl.ds(0, CW)],
                chunk_v.at[:, :, pl.ds(0, CW)], sem).wait()

        def run_window(p, u, win, half):
            cstart = jnp.minimum(win, jnp.int32(TAILC)) * 128
            cend = cstart + CW

            def inner_body(carry):
                pp, uu = carry
                extract(pp, uu, cstart, half)
                nxt = seg_v[pl.ds(pp + 1, 16)]
                return pp + 1, nxt[0]

            def inner_cond(carry):
                pp, uu = carry
                return (pp < BPW) & ((uu < cend) | (uu >= TAILROW))

            return lax.while_loop(inner_cond, inner_body, (p, u))

        head = seg_v[pl.ds(0, 16)]
        u0 = head[0]
        u_last = seg_v[pl.ds(BPW - 16, 16)][15]
        tc_lo = jnp.minimum(u0 >> 7, jnp.int32(TAILC))
        span_hi = jnp.minimum(u_last >> 7, jnp.int32(NTC - 1))
        nchunks = jnp.maximum((span_hi - tc_lo + CHW) // CHW, 1)
        npairs = (nchunks + 1) // 2

        fetch(tc_lo, 0, semA)
        fetch(tc_lo + CHW, 1, semB)

        def pair_body(k2, carry):
            p, u = carry
            drain(semA)
            p, u = run_window(p, u, tc_lo + (2 * k2) * CHW, 0)
            fetch(tc_lo + (2 * k2 + 2) * CHW, 0, semA)
            drain(semB)
            p, u = run_window(p, u, tc_lo + (2 * k2 + 1) * CHW, 1)
            fetch(tc_lo + (2 * k2 + 3) * CHW, 1, semB)
            return p, u

        lax.fori_loop(0, npairs, pair_body, (jnp.int32(0), u0))
        drain(semA)
        drain(semB)

    process_table(suidx_hbm, uperm_hbm, utab_hbm, utail_hbm, uscr_hbm)
    process_table(siidx_hbm, iperm_hbm, itab_hbm, itail_hbm, iscr_hbm)


def _dot_body(u_ref, i_ref, o_ref):
    prod = u_ref[...] * i_ref[...]
    col = lax.broadcasted_iota(jnp.int32, prod.shape, 1)
    prod = jnp.where(col < D, prod, 0.0)
    o_ref[...] = jnp.sum(prod, axis=1)


_dot_tc = pl.pallas_call(
    _dot_body,
    grid=(8,),
    in_specs=[pl.BlockSpec((BATCH // 8, 128), lambda i: (i, 0)),
              pl.BlockSpec((BATCH // 8, 128), lambda i: (i, 0))],
    out_specs=pl.BlockSpec((BATCH // 8,), lambda i: (i,)),
    out_shape=jax.ShapeDtypeStruct((BATCH,), jnp.float32),
)


def kernel(user_indices, item_indices, user_table, item_table):
    ui = user_indices.astype(jnp.int32)
    ii = item_indices.astype(jnp.int32)
    pos = lax.iota(jnp.int32, BATCH)
    su, pu = lax.sort_key_val(ui, pos)
    si, pi = lax.sort_key_val(ii, pos)
    utab3 = user_table.T.reshape(8, 8, NROWS)
    itab3 = item_table.T.reshape(8, 8, NROWS)
    utail = jnp.pad(user_table[TAILROW:].T, ((0, 0), (0, 128 - (NROWS - TAILROW))))
    itail = jnp.pad(item_table[TAILROW:].T, ((0, 0), (0, 128 - (NROWS - TAILROW))))
    uscr, iscr = _gather_kernel(su, pu, si, pi, utab3, itab3, utail, itail)
    out = _dot_tc(uscr, iscr)
    return out.reshape(BATCH, 1)
